# Initial kernel scaffold; baseline (speedup 1.0000x reference)
import jax
import jax.numpy as jnp
from jax.experimental import pallas as pl


def kernel(x, table):
    raise NotImplementedError("write your pallas kernel here")


# speedup vs baseline: 3.3237x; 3.3237x over previous
import functools

import jax
import jax.numpy as jnp
from jax import lax
from jax.experimental import pallas as pl
from jax.experimental.pallas import tpu as pltpu
from jax.experimental.pallas import tpu_sc as plsc

INP_SIZE = 1000000
HIDDEN = 32
WEIGHTS = (0.1, 0.2, 0.4, 0.2, 0.1)
CHUNK = 128
LANES = 16


def _body(x_hbm, table_hbm, out_hbm, xb, idxs, rows, outb, sem):
    n_tok = x_hbm.shape[0]
    n_workers = 32
    per_w = n_tok // n_workers
    n_chunks = per_w // CHUNK

    wid = lax.axis_index("s") * 2 + lax.axis_index("c")
    base = wid * per_w

    def chunk_body(c, carry):
        tok = base + c * CHUNK
        pltpu.sync_copy(x_hbm.at[pl.ds(tok, CHUNK)], xb)

        for v in range(CHUNK // LANES):
            t = xb[pl.ds(v * LANES, LANES)]
            for s in range(5):
                u = jnp.clip(t + (s - 2), 0, INP_SIZE - 1)
                idxs[s, pl.ds(v * LANES, LANES)] = u

        copies = [
            pltpu.async_copy(table_hbm.at[idxs.at[s]], rows.at[s], sem)
            for s in range(5)
        ]
        for cp in copies:
            cp.wait()

        def sum_body(j, carry2):
            for half in range(HIDDEN // LANES):
                ln = pl.ds(half * LANES, LANES)
                r0 = rows[0, j, ln]
                r1 = rows[1, j, ln]
                r2 = rows[2, j, ln]
                r3 = rows[3, j, ln]
                r4 = rows[4, j, ln]
                outb[j, ln] = (
                    WEIGHTS[0] * (r0 + r4)
                    + WEIGHTS[1] * (r1 + r3)
                    + WEIGHTS[2] * r2
                )
            return carry2

        lax.fori_loop(0, CHUNK, sum_body, 0)

        pltpu.sync_copy(outb, out_hbm.at[pl.ds(tok, CHUNK)])
        return carry

    lax.fori_loop(0, n_chunks, chunk_body, 0)


def kernel(x, table):
    b, t = x.shape
    n_tok = b * t
    xf = x.reshape(n_tok).astype(jnp.int32)

    mesh = plsc.VectorSubcoreMesh(core_axis_name="c", subcore_axis_name="s")
    run = functools.partial(
        pl.kernel,
        mesh=mesh,
        out_type=jax.ShapeDtypeStruct((n_tok, HIDDEN), jnp.float32),
        scratch_types=[
            pltpu.VMEM((CHUNK,), jnp.int32),
            pltpu.VMEM((5, CHUNK), jnp.int32),
            pltpu.VMEM((5, CHUNK, HIDDEN), jnp.float32),
            pltpu.VMEM((CHUNK, HIDDEN), jnp.float32),
            pltpu.SemaphoreType.DMA,
        ],
        compiler_params=pltpu.CompilerParams(use_tc_tiling_on_sc=False),
    )(_body)

    out = run(xf, table)
    return out.reshape(b, t, HIDDEN)


# speedup vs baseline: 3.7367x; 1.1243x over previous
import functools

import jax
import jax.numpy as jnp
from jax import lax
from jax.experimental import pallas as pl
from jax.experimental.pallas import tpu as pltpu
from jax.experimental.pallas import tpu_sc as plsc

INP_SIZE = 1000000
HIDDEN = 32
W0, W1, W2 = 0.1, 0.2, 0.4
CHUNK = 128
LANES = 16
N_WORKERS = 32


def _body(x_hbm, table_hbm, out_hbm, xall, idxs, rows, outb, sg0, sg1, so):
    n_tok = x_hbm.shape[0]
    per_w = n_tok // N_WORKERS
    n_chunks = per_w // CHUNK

    wid = lax.axis_index("s") * 2 + lax.axis_index("c")
    base = wid * per_w

    pltpu.sync_copy(x_hbm.at[pl.ds(base, per_w)], xall)

    def build_idx(c, slot):
        @plsc.parallel_loop(0, CHUNK // LANES)
        def _(v):
            t = xall[pl.ds(c * CHUNK + v * LANES, LANES)]
            for s in range(5):
                u = jnp.clip(t + (s - 2), 0, INP_SIZE - 1)
                idxs[slot, s, pl.ds(v * LANES, LANES)] = u

    def fire_gathers(slot, sem):
        for s in range(5):
            pltpu.async_copy(table_hbm.at[idxs.at[slot, s]], rows.at[slot, s], sem)

    def drain_gathers(slot, sem):
        for s in range(5):
            pltpu.make_async_copy(
                table_hbm.at[idxs.at[slot, s]], rows.at[slot, s], sem
            ).wait()

    def wait_out():
        pltpu.make_async_copy(outb.at[0], out_hbm.at[pl.ds(0, CHUNK)], so).wait()

    def compute(c, slot):
        @plsc.parallel_loop(0, CHUNK, unroll=4)
        def _(j):
            for half in range(HIDDEN // LANES):
                ln = pl.ds(half * LANES, LANES)
                r0 = rows[slot, 0, j, ln]
                r1 = rows[slot, 1, j, ln]
                r2 = rows[slot, 2, j, ln]
                r3 = rows[slot, 3, j, ln]
                r4 = rows[slot, 4, j, ln]
                outb[slot, j, ln] = W0 * (r0 + r4) + W1 * (r1 + r3) + W2 * r2

        pltpu.async_copy(
            outb.at[slot], out_hbm.at[pl.ds(base + c * CHUNK, CHUNK)], so
        )

    build_idx(0, 0)
    fire_gathers(0, sg0)

    def pair_body(k, carry):
        c0 = 2 * k
        c1 = c0 + 1

        build_idx(c1, 1)
        fire_gathers(1, sg1)

        drain_gathers(0, sg0)

        @pl.when(k > 0)
        def _():
            wait_out()

        compute(c0, 0)

        @pl.when(k < n_chunks // 2 - 1)
        def _():
            build_idx(c0 + 2, 0)
            fire_gathers(0, sg0)

        drain_gathers(1, sg1)

        @pl.when(k > 0)
        def _():
            wait_out()

        compute(c1, 1)
        return carry

    lax.fori_loop(0, n_chunks // 2, pair_body, 0)

    wait_out()
    wait_out()


def kernel(x, table):
    b, t = x.shape
    n_tok = b * t
    per_w = n_tok // N_WORKERS
    xf = x.reshape(n_tok).astype(jnp.int32)

    mesh = plsc.VectorSubcoreMesh(core_axis_name="c", subcore_axis_name="s")
    run = functools.partial(
        pl.kernel,
        mesh=mesh,
        out_type=jax.ShapeDtypeStruct((n_tok, HIDDEN), jnp.float32),
        scratch_types=[
            pltpu.VMEM((per_w,), jnp.int32),
            pltpu.VMEM((2, 5, CHUNK), jnp.int32),
            pltpu.VMEM((2, 5, CHUNK, HIDDEN), jnp.float32),
            pltpu.VMEM((2, CHUNK, HIDDEN), jnp.float32),
            pltpu.SemaphoreType.DMA,
            pltpu.SemaphoreType.DMA,
            pltpu.SemaphoreType.DMA,
        ],
        compiler_params=pltpu.CompilerParams(use_tc_tiling_on_sc=False),
    )(_body)

    out = run(xf, table)
    return out.reshape(b, t, HIDDEN)
